# baseline (speedup 1.0000x reference)
import functools

import jax
import jax.numpy as jnp
from jax import lax
from jax.experimental import pallas as pl
from jax.experimental.pallas import tpu as pltpu
from jax.experimental.pallas import tpu_sc as plsc

N = 10000
E = 320000
D_IN = 128
D_HID = 128
D_OUT = 40

NC, NS = 2, 16
NW = NC * NS
CHUNK = 128
CPW = 80
EPAD = NW * CPW * CHUNK
NP = 10112
RPT = NP // NS

D1 = 128
DDEG = 32
D2 = 64

_mesh = plsc.VectorSubcoreMesh(core_axis_name="c", subcore_axis_name="s")


def _make_seg_sum(d, dtype, with_deg):
    lanes = 16 * (4 // jnp.dtype(dtype).itemsize)

    out_type = [jax.ShapeDtypeStruct((2 * NC, NP, d), dtype)]
    scratch = [
        pltpu.VMEM((4, 2, CHUNK), jnp.int32),
        pltpu.VMEM((CHUNK, d), dtype),
        pltpu.VMEM((CHUNK, d), dtype),
        pltpu.VMEM_SHARED((2, NP, d), dtype),
        pltpu.SemaphoreType.DMA,
        pltpu.SemaphoreType.DMA,
        pltpu.SemaphoreType.DMA,
        pltpu.SemaphoreType.DMA,
        pltpu.SemaphoreType.DMA,
        pltpu.SemaphoreType.DMA,
    ]
    if with_deg:
        out_type.append(jax.ShapeDtypeStruct((NC, NP, DDEG), dtype))
        scratch.append(pltpu.VMEM((CHUNK, DDEG), dtype))
        scratch.append(pltpu.VMEM_SHARED((NP, DDEG), dtype))
        scratch.append(pltpu.SemaphoreType.DMA)

    @functools.partial(
        pl.kernel,
        mesh=_mesh,
        compiler_params=pltpu.CompilerParams(use_tc_tiling_on_sc=False),
        out_type=out_type,
        scratch_types=scratch,
    )
    def seg_sum(vals_hbm, idx_hbm, out_hbm, *rest):
        if with_deg:
            (deg_hbm, idxr, rows0, rows1, accab,
             isem0, isem1, isem2, isem3, gsem0, gsem1,
             onesb, dacc, dsem) = rest
        else:
            (idxr, rows0, rows1, accab,
             isem0, isem1, isem2, isem3, gsem0, gsem1) = rest
        c = lax.axis_index("c")
        s = lax.axis_index("s")
        w = s * NC + c
        base = pl.multiple_of(w * CPW, 8)
        roff = pl.multiple_of(s * RPT, 8)
        isems = (isem0, isem1, isem2, isem3)
        gsems = (gsem0, gsem1)
        bufs = (rows0, rows1)
        accs = (accab.at[0], accab.at[1])

        def zrow(r, carry):
            def zcol(kk, carry2):
                rows0[r, pl.ds(kk * lanes, lanes)] = jnp.zeros((lanes,), dtype)
                return carry2
            return lax.fori_loop(0, d // lanes, zcol, carry)

        lax.fori_loop(0, CHUNK, zrow, jnp.int32(0))
        nfull, rem = RPT // CHUNK, RPT % CHUNK

        def zfill(dst_acc):
            for k in range(nfull):
                pltpu.sync_copy(rows0, dst_acc.at[pl.ds(roff + k * CHUNK, CHUNK)])
            if rem:
                pltpu.sync_copy(rows0.at[pl.ds(0, rem)],
                                dst_acc.at[pl.ds(roff + nfull * CHUNK, rem)])

        for acc in accs:
            zfill(acc)
        if with_deg:
            dlanes = 32
            def dzrow(r, carry):
                def dzcol(kk, carry2):
                    onesb[r, pl.ds(kk * dlanes, dlanes)] = jnp.zeros(
                        (dlanes,), dtype)
                    return carry2
                return lax.fori_loop(0, DDEG // dlanes, dzcol, carry)
            lax.fori_loop(0, CHUNK, dzrow, jnp.int32(0))
            for k in range(nfull):
                pltpu.sync_copy(onesb, dacc.at[pl.ds(roff + k * CHUNK, CHUNK)])
            if rem:
                pltpu.sync_copy(onesb.at[pl.ds(0, rem)],
                                dacc.at[pl.ds(roff + nfull * CHUNK, rem)])
            def orow(r, carry):
                def ocol(kk, carry2):
                    onesb[r, pl.ds(kk * dlanes, dlanes)] = jnp.ones(
                        (dlanes,), dtype)
                    return carry2
                return lax.fori_loop(0, DDEG // dlanes, ocol, carry)
            lax.fori_loop(0, CHUNK, orow, jnp.int32(0))
        plsc.subcore_barrier()

        def fetch_idx(cf, slot, sem):
            row = jnp.where(cf < CPW, base + cf, base)
            pltpu.async_copy(idx_hbm.at[row], idxr.at[slot], sem)

        for k in range(4):
            fetch_idx(jnp.int32(k), k, isems[k])
        for b in range(2):
            pltpu.make_async_copy(idx_hbm.at[base], idxr.at[b], isems[b]).wait()
            pltpu.async_copy(vals_hbm.at[idxr.at[b, 0]], bufs[b], gsems[b])

        def body(i, carry):
            for k in range(4):
                j = 4 * i + k
                b = k % 2
                k2 = (k + 2) % 4
                pltpu.make_async_copy(vals_hbm.at[idxr.at[k, 0]],
                                      bufs[b], gsems[b]).wait()
                pltpu.sync_copy(bufs[b], accs[b].at[idxr.at[k, 1]], add=True)
                if with_deg:
                    pltpu.async_copy(onesb, dacc.at[idxr.at[k, 1]], dsem,
                                     add=True)
                fetch_idx(j + 4, k, isems[k])
                pltpu.make_async_copy(idx_hbm.at[base],
                                      idxr.at[k2], isems[k2]).wait()
                pltpu.async_copy(vals_hbm.at[idxr.at[k2, 0]], bufs[b], gsems[b])
            return carry

        lax.fori_loop(0, CPW // 4, body, 0)
        if with_deg:
            def ddrain(_, carry):
                pltpu.make_async_copy(onesb, dacc.at[pl.ds(0, CHUNK)],
                                      dsem).wait()
                return carry
            lax.fori_loop(0, CPW, ddrain, jnp.int32(0))
        for b in range(2):
            pltpu.make_async_copy(vals_hbm.at[idxr.at[b, 0]],
                                  bufs[b], gsems[b]).wait()
        for k in (2, 3):
            pltpu.make_async_copy(idx_hbm.at[base], idxr.at[k], isems[k]).wait()
        plsc.subcore_barrier()
        for a in range(2):
            pltpu.sync_copy(accs[a].at[pl.ds(roff, RPT)],
                            out_hbm.at[2 * c + a, pl.ds(roff, RPT)])
        if with_deg:
            pltpu.sync_copy(dacc.at[pl.ds(roff, RPT)],
                            deg_hbm.at[c, pl.ds(roff, RPT)])

    return seg_sum


_seg_sum_d1 = _make_seg_sum(D1, jnp.bfloat16, True)
_seg_sum_d2 = _make_seg_sum(D2, jnp.bfloat16, False)

BLK = 1000


def _tc1_body(x_ref, aggp_ref, degp_ref, ws1_ref, wn1_ref, b1_ref, wn2e_ref,
              h1_ref, z_ref, r_ref):
    srow = (aggp_ref[0].astype(jnp.float32)
            + aggp_ref[1].astype(jnp.float32)
            + aggp_ref[2].astype(jnp.float32)
            + aggp_ref[3].astype(jnp.float32))
    degs = (degp_ref[0].astype(jnp.float32)
            + degp_ref[1].astype(jnp.float32))
    deg = degs[:, :1]
    recip = 1.0 / jnp.maximum(deg, 1.0)
    mean_c = jnp.dot(srow * recip, wn1_ref[...],
                     preferred_element_type=jnp.float32)
    h = jnp.dot(x_ref[...], ws1_ref[...],
                preferred_element_type=jnp.float32) + mean_c + b1_ref[...]
    h = jnp.maximum(h, 0.0)
    h1_ref[...] = h
    z_ref[...] = jnp.dot(h, wn2e_ref[...],
                         preferred_element_type=jnp.float32).astype(jnp.bfloat16)
    r_ref[...] = recip


def _tc2_body(h1_ref, agg2p_ref, r_ref, ws2e_ref, b2e_ref, out_ref):
    mean2 = (agg2p_ref[0].astype(jnp.float32)
             + agg2p_ref[1].astype(jnp.float32)
             + agg2p_ref[2].astype(jnp.float32)
             + agg2p_ref[3].astype(jnp.float32)) * r_ref[...]
    o = jnp.dot(h1_ref[...], ws2e_ref[...],
                preferred_element_type=jnp.float32) + mean2 + b2e_ref[...]
    out_ref[...] = o[:, :D_OUT]


_tc1 = pl.pallas_call(
    _tc1_body,
    grid=(N // BLK,),
    in_specs=[
        pl.BlockSpec((BLK, D_IN), lambda i: (i, 0)),
        pl.BlockSpec((2 * NC, BLK, D1), lambda i: (0, i, 0)),
        pl.BlockSpec((NC, BLK, DDEG), lambda i: (0, i, 0)),
        pl.BlockSpec((D_IN, D_HID), lambda i: (0, 0)),
        pl.BlockSpec((D1, D_HID), lambda i: (0, 0)),
        pl.BlockSpec((1, D_HID), lambda i: (0, 0)),
        pl.BlockSpec((D_HID, D2), lambda i: (0, 0)),
    ],
    out_specs=[
        pl.BlockSpec((BLK, D_HID), lambda i: (i, 0)),
        pl.BlockSpec((BLK, D2), lambda i: (i, 0)),
        pl.BlockSpec((BLK, 1), lambda i: (i, 0)),
    ],
    out_shape=[
        jax.ShapeDtypeStruct((N, D_HID), jnp.float32),
        jax.ShapeDtypeStruct((N, D2), jnp.bfloat16),
        jax.ShapeDtypeStruct((N, 1), jnp.float32),
    ],
)

_tc2 = pl.pallas_call(
    _tc2_body,
    grid=(N // BLK,),
    in_specs=[
        pl.BlockSpec((BLK, D_HID), lambda i: (i, 0)),
        pl.BlockSpec((2 * NC, BLK, D2), lambda i: (0, i, 0)),
        pl.BlockSpec((BLK, 1), lambda i: (i, 0)),
        pl.BlockSpec((D_HID, D2), lambda i: (0, 0)),
        pl.BlockSpec((1, D2), lambda i: (0, 0)),
    ],
    out_specs=pl.BlockSpec((BLK, D_OUT), lambda i: (i, 0)),
    out_shape=jax.ShapeDtypeStruct((N, D_OUT), jnp.float32),
)


def kernel(features, edge_index, W_self1, W_neigh1, b1, W_self2, W_neigh2, b2):
    src = edge_index[0].astype(jnp.int32)
    dst = edge_index[1].astype(jnp.int32)
    src2d = jnp.pad(src, (0, EPAD - E)).reshape(NW * CPW, CHUNK)
    dst2d = jnp.pad(dst, (0, EPAD - E), constant_values=N).reshape(NW * CPW, CHUNK)
    idx_comb = jnp.stack([src2d, dst2d], axis=1)

    xb = features.astype(jnp.bfloat16)

    aggp, degp = _seg_sum_d1(xb, idx_comb)

    wn2e = jnp.concatenate(
        [W_neigh2, jnp.zeros((D_HID, D2 - D_OUT), jnp.float32)], axis=1)
    h1, z, recip = _tc1(features, aggp, degp, W_self1, W_neigh1,
                        b1.reshape(1, D_HID), wn2e)

    (agg2p,) = _seg_sum_d2(z, idx_comb)

    ws2e = jnp.concatenate(
        [W_self2, jnp.zeros((D_HID, D2 - D_OUT), jnp.float32)], axis=1)
    b2e = jnp.concatenate(
        [b2, jnp.zeros((D2 - D_OUT,), jnp.float32)]).reshape(1, D2)
    return _tc2(h1, agg2p, recip, ws2e, b2e)


# speedup vs baseline: 1.0154x; 1.0154x over previous
import functools

import jax
import jax.numpy as jnp
from jax import lax
from jax.experimental import pallas as pl
from jax.experimental.pallas import tpu as pltpu
from jax.experimental.pallas import tpu_sc as plsc

N = 10000
E = 320000
D_IN = 128
D_HID = 128
D_OUT = 40

NC, NS = 2, 16
NW = NC * NS
CHUNK = 128
CPW = 80
EPAD = NW * CPW * CHUNK
NP = 10112
RPT = NP // NS

D1 = 128
DDEG = 32
D2 = 64

_mesh = plsc.VectorSubcoreMesh(core_axis_name="c", subcore_axis_name="s")


def _make_seg_sum(d, dtype, with_deg):
    lanes = 16 * (4 // jnp.dtype(dtype).itemsize)

    out_type = [jax.ShapeDtypeStruct((2 * NC, NP, d), dtype)]
    scratch = [
        pltpu.VMEM((4, 2, CHUNK), jnp.int32),
        pltpu.VMEM((CHUNK, d), dtype),
        pltpu.VMEM((CHUNK, d), dtype),
        pltpu.VMEM_SHARED((2, NP, d), dtype),
        pltpu.SemaphoreType.DMA,
        pltpu.SemaphoreType.DMA,
        pltpu.SemaphoreType.DMA,
        pltpu.SemaphoreType.DMA,
        pltpu.SemaphoreType.DMA,
        pltpu.SemaphoreType.DMA,
    ]
    if with_deg:
        out_type.append(jax.ShapeDtypeStruct((NC, NP, DDEG), dtype))
        scratch.append(pltpu.VMEM((CHUNK, DDEG), dtype))
        scratch.append(pltpu.VMEM_SHARED((NP, DDEG), dtype))

    @functools.partial(
        pl.kernel,
        mesh=_mesh,
        compiler_params=pltpu.CompilerParams(use_tc_tiling_on_sc=False),
        out_type=out_type,
        scratch_types=scratch,
    )
    def seg_sum(vals_hbm, idx_hbm, out_hbm, *rest):
        if with_deg:
            (deg_hbm, idxr, rows0, rows1, accab,
             isem0, isem1, isem2, isem3, gsem0, gsem1, onesb, dacc) = rest
        else:
            (idxr, rows0, rows1, accab,
             isem0, isem1, isem2, isem3, gsem0, gsem1) = rest
        c = lax.axis_index("c")
        s = lax.axis_index("s")
        w = s * NC + c
        base = pl.multiple_of(w * CPW, 8)
        roff = pl.multiple_of(s * RPT, 8)
        isems = (isem0, isem1, isem2, isem3)
        gsems = (gsem0, gsem1)
        bufs = (rows0, rows1)
        accs = (accab.at[0], accab.at[1])

        def zrow(r, carry):
            def zcol(kk, carry2):
                rows0[r, pl.ds(kk * lanes, lanes)] = jnp.zeros((lanes,), dtype)
                return carry2
            return lax.fori_loop(0, d // lanes, zcol, carry)

        lax.fori_loop(0, CHUNK, zrow, jnp.int32(0))
        nfull, rem = RPT // CHUNK, RPT % CHUNK

        def zfill(dst_acc):
            for k in range(nfull):
                pltpu.sync_copy(rows0, dst_acc.at[pl.ds(roff + k * CHUNK, CHUNK)])
            if rem:
                pltpu.sync_copy(rows0.at[pl.ds(0, rem)],
                                dst_acc.at[pl.ds(roff + nfull * CHUNK, rem)])

        for acc in accs:
            zfill(acc)
        if with_deg:
            dlanes = 32
            def dzrow(r, carry):
                def dzcol(kk, carry2):
                    onesb[r, pl.ds(kk * dlanes, dlanes)] = jnp.zeros(
                        (dlanes,), dtype)
                    return carry2
                return lax.fori_loop(0, DDEG // dlanes, dzcol, carry)
            lax.fori_loop(0, CHUNK, dzrow, jnp.int32(0))
            for k in range(nfull):
                pltpu.sync_copy(onesb, dacc.at[pl.ds(roff + k * CHUNK, CHUNK)])
            if rem:
                pltpu.sync_copy(onesb.at[pl.ds(0, rem)],
                                dacc.at[pl.ds(roff + nfull * CHUNK, rem)])
            def orow(r, carry):
                def ocol(kk, carry2):
                    onesb[r, pl.ds(kk * dlanes, dlanes)] = jnp.ones(
                        (dlanes,), dtype)
                    return carry2
                return lax.fori_loop(0, DDEG // dlanes, ocol, carry)
            lax.fori_loop(0, CHUNK, orow, jnp.int32(0))
        plsc.subcore_barrier()

        def fetch_idx(cf, slot, sem):
            row = jnp.where(cf < CPW, base + cf, base)
            pltpu.async_copy(idx_hbm.at[row], idxr.at[slot], sem)

        for k in range(4):
            fetch_idx(jnp.int32(k), k, isems[k])
        for b in range(2):
            pltpu.make_async_copy(idx_hbm.at[base], idxr.at[b], isems[b]).wait()
            pltpu.async_copy(vals_hbm.at[idxr.at[b, 0]], bufs[b], gsems[b])

        def body(i, carry):
            for k in range(4):
                j = 4 * i + k
                b = k % 2
                k2 = (k + 2) % 4
                pltpu.make_async_copy(vals_hbm.at[idxr.at[k, 0]],
                                      bufs[b], gsems[b]).wait()
                pltpu.sync_copy(bufs[b], accs[b].at[idxr.at[k, 1]], add=True)
                if with_deg:
                    pltpu.sync_copy(onesb, dacc.at[idxr.at[k, 1]], add=True)
                fetch_idx(j + 4, k, isems[k])
                pltpu.make_async_copy(idx_hbm.at[base],
                                      idxr.at[k2], isems[k2]).wait()
                pltpu.async_copy(vals_hbm.at[idxr.at[k2, 0]], bufs[b], gsems[b])
            return carry

        lax.fori_loop(0, CPW // 4, body, 0)
        for b in range(2):
            pltpu.make_async_copy(vals_hbm.at[idxr.at[b, 0]],
                                  bufs[b], gsems[b]).wait()
        for k in (2, 3):
            pltpu.make_async_copy(idx_hbm.at[base], idxr.at[k], isems[k]).wait()
        plsc.subcore_barrier()
        for a in range(2):
            pltpu.sync_copy(accs[a].at[pl.ds(roff, RPT)],
                            out_hbm.at[2 * c + a, pl.ds(roff, RPT)])
        if with_deg:
            pltpu.sync_copy(dacc.at[pl.ds(roff, RPT)],
                            deg_hbm.at[c, pl.ds(roff, RPT)])

    return seg_sum


_seg_sum_d1 = _make_seg_sum(D1, jnp.bfloat16, True)
_seg_sum_d2 = _make_seg_sum(D2, jnp.bfloat16, False)

BLK = 1000


def _tc1_body(x_ref, aggp_ref, degp_ref, ws1_ref, wn1_ref, b1_ref, wn2e_ref,
              h1_ref, z_ref, r_ref):
    srow = (aggp_ref[0].astype(jnp.float32)
            + aggp_ref[1].astype(jnp.float32)
            + aggp_ref[2].astype(jnp.float32)
            + aggp_ref[3].astype(jnp.float32))
    degs = (degp_ref[0].astype(jnp.float32)
            + degp_ref[1].astype(jnp.float32))
    deg = degs[:, :1]
    recip = 1.0 / jnp.maximum(deg, 1.0)
    mean_c = jnp.dot(srow * recip, wn1_ref[...],
                     preferred_element_type=jnp.float32)
    h = jnp.dot(x_ref[...], ws1_ref[...],
                preferred_element_type=jnp.float32) + mean_c + b1_ref[...]
    h = jnp.maximum(h, 0.0)
    h1_ref[...] = h
    z_ref[...] = jnp.dot(h, wn2e_ref[...],
                         preferred_element_type=jnp.float32).astype(jnp.bfloat16)
    r_ref[...] = recip


def _tc2_body(h1_ref, agg2p_ref, r_ref, ws2e_ref, b2e_ref, out_ref):
    mean2 = (agg2p_ref[0].astype(jnp.float32)
             + agg2p_ref[1].astype(jnp.float32)
             + agg2p_ref[2].astype(jnp.float32)
             + agg2p_ref[3].astype(jnp.float32)) * r_ref[...]
    o = jnp.dot(h1_ref[...], ws2e_ref[...],
                preferred_element_type=jnp.float32) + mean2 + b2e_ref[...]
    out_ref[...] = o[:, :D_OUT]


_tc1 = pl.pallas_call(
    _tc1_body,
    grid=(N // BLK,),
    in_specs=[
        pl.BlockSpec((BLK, D_IN), lambda i: (i, 0)),
        pl.BlockSpec((2 * NC, BLK, D1), lambda i: (0, i, 0)),
        pl.BlockSpec((NC, BLK, DDEG), lambda i: (0, i, 0)),
        pl.BlockSpec((D_IN, D_HID), lambda i: (0, 0)),
        pl.BlockSpec((D1, D_HID), lambda i: (0, 0)),
        pl.BlockSpec((1, D_HID), lambda i: (0, 0)),
        pl.BlockSpec((D_HID, D2), lambda i: (0, 0)),
    ],
    out_specs=[
        pl.BlockSpec((BLK, D_HID), lambda i: (i, 0)),
        pl.BlockSpec((BLK, D2), lambda i: (i, 0)),
        pl.BlockSpec((BLK, 1), lambda i: (i, 0)),
    ],
    out_shape=[
        jax.ShapeDtypeStruct((N, D_HID), jnp.float32),
        jax.ShapeDtypeStruct((N, D2), jnp.bfloat16),
        jax.ShapeDtypeStruct((N, 1), jnp.float32),
    ],
)

_tc2 = pl.pallas_call(
    _tc2_body,
    grid=(N // BLK,),
    in_specs=[
        pl.BlockSpec((BLK, D_HID), lambda i: (i, 0)),
        pl.BlockSpec((2 * NC, BLK, D2), lambda i: (0, i, 0)),
        pl.BlockSpec((BLK, 1), lambda i: (i, 0)),
        pl.BlockSpec((D_HID, D2), lambda i: (0, 0)),
        pl.BlockSpec((1, D2), lambda i: (0, 0)),
    ],
    out_specs=pl.BlockSpec((BLK, D_OUT), lambda i: (i, 0)),
    out_shape=jax.ShapeDtypeStruct((N, D_OUT), jnp.float32),
)


def kernel(features, edge_index, W_self1, W_neigh1, b1, W_self2, W_neigh2, b2):
    src = edge_index[0].astype(jnp.int32)
    dst = edge_index[1].astype(jnp.int32)
    src2d = jnp.pad(src, (0, EPAD - E)).reshape(NW * CPW, CHUNK)
    dst2d = jnp.pad(dst, (0, EPAD - E), constant_values=N).reshape(NW * CPW, CHUNK)
    idx_comb = jnp.stack([src2d, dst2d], axis=1)

    xb = features.astype(jnp.bfloat16)

    aggp, degp = _seg_sum_d1(xb, idx_comb)

    wn2e = jnp.concatenate(
        [W_neigh2, jnp.zeros((D_HID, D2 - D_OUT), jnp.float32)], axis=1)
    h1, z, recip = _tc1(features, aggp, degp, W_self1, W_neigh1,
                        b1.reshape(1, D_HID), wn2e)

    (agg2p,) = _seg_sum_d2(z, idx_comb)

    ws2e = jnp.concatenate(
        [W_self2, jnp.zeros((D_HID, D2 - D_OUT), jnp.float32)], axis=1)
    b2e = jnp.concatenate(
        [b2, jnp.zeros((D2 - D_OUT,), jnp.float32)]).reshape(1, D2)
    return _tc2(h1, agg2p, recip, ws2e, b2e)
